# baseline (speedup 1.0000x reference)
import functools

import jax
import jax.numpy as jnp
from jax import lax
from jax.experimental import pallas as pl
from jax.experimental.pallas import tpu as pltpu
from jax.experimental.pallas import tpu_sc as plsc

N = 10000
E = 320000
D = 128
H = 128

NC = 2
NS = 16
NW = NC * NS
EPW = E // NW
CH = 80
SCH = 25
NSC = EPW // (CH * SCH)
ZR = N // 10
NP = 10240
HS = NP // 10

_MESH = plsc.VectorSubcoreMesh(
    core_axis_name="c", subcore_axis_name="s", num_cores=NC, num_subcores=NS
)


def _sc_aggr_body(with_ef, feat_hbm, src_hbm, dst_hbm, ef_hbm, z2_hbm, z1_hbm,
                  hn_out, he0_out, he1_out, src_v, dst_v, ef_v, rows0, rows1,
                  he_v, acc_sh, he_sh, g0, g1, s0, s1):
    c = lax.axis_index("c")
    s = lax.axis_index("s")
    wid = s * NC + c

    @pl.when(s < 10)
    def _zero():
        pltpu.sync_copy(z2_hbm, acc_sh.at[pl.ds(s * ZR, ZR)])
        if with_ef:
            pltpu.sync_copy(z1_hbm.at[pl.ds(s * HS, HS)], he_v)
            pltpu.sync_copy(he_v, he_sh.at[pl.ds(s * HS, HS)])

    plsc.subcore_barrier()

    def gather_start(j, buf, sem):
        pltpu.make_async_copy(feat_hbm.at[src_v.at[j]], buf, sem).start()

    def gather_wait(j, buf, sem):
        pltpu.make_async_copy(feat_hbm.at[src_v.at[j]], buf, sem).wait()

    def scatter_start(j, buf, sem):
        pltpu.async_copy(buf, acc_sh.at[dst_v.at[j]], sem, add=True)

    def scatter_wait(j, buf, sem):
        pltpu.make_async_copy(buf, acc_sh.at[dst_v.at[j]], sem).wait()

    def he_scatter(j):
        if with_ef:
            pltpu.sync_copy(ef_v.at[j], he_sh.at[dst_v.at[j]], add=True)

    def super_step(k, carry):
        pltpu.sync_copy(src_hbm.at[wid * NSC + k], src_v)
        pltpu.sync_copy(dst_hbm.at[wid * NSC + k], dst_v)
        if with_ef:
            pltpu.sync_copy(ef_hbm.at[wid * NSC + k], ef_v)

        gather_start(0, rows0, g0)
        gather_start(1, rows1, g1)

        def pair(jj, inner):
            j0 = 2 * jj
            gather_wait(j0, rows0, g0)
            scatter_start(j0, rows0, s0)
            he_scatter(j0)
            gather_wait(j0 + 1, rows1, g1)
            scatter_start(j0 + 1, rows1, s1)
            he_scatter(j0 + 1)
            scatter_wait(j0, rows0, s0)
            gather_start(j0 + 2, rows0, g0)
            scatter_wait(j0 + 1, rows1, s1)

            @pl.when(j0 + 3 < SCH)
            def _():
                gather_start(j0 + 3, rows1, g1)

            return inner

        lax.fori_loop(0, SCH // 2, pair, 0)
        gather_wait(SCH - 1, rows0, g0)
        pltpu.sync_copy(rows0, acc_sh.at[dst_v.at[SCH - 1]], add=True)
        he_scatter(SCH - 1)
        return carry

    lax.fori_loop(0, NSC, super_step, 0)
    plsc.subcore_barrier()

    @pl.when(s < 10)
    def _writeback():
        pltpu.sync_copy(acc_sh.at[pl.ds(s * ZR, ZR)], hn_out.at[c, pl.ds(s * ZR, ZR)])
        if with_ef:
            pltpu.sync_copy(he_sh.at[pl.ds(s * HS, HS)], he_v)

            @pl.when(c == 0)
            def _wb_he0():
                pltpu.sync_copy(he_v, he0_out.at[pl.ds(s * HS, HS)])

            @pl.when(c == 1)
            def _wb_he1():
                pltpu.sync_copy(he_v, he1_out.at[pl.ds(s * HS, HS)])


def _make_sc_aggr(with_ef):
    return pl.kernel(
        functools.partial(_sc_aggr_body, with_ef),
        out_type=[
            jax.ShapeDtypeStruct((NC, N, D), jnp.float32),
            jax.ShapeDtypeStruct((NP,), jnp.float32),
            jax.ShapeDtypeStruct((NP,), jnp.float32),
        ],
        mesh=_MESH,
        scratch_types=[
            pltpu.VMEM((SCH, CH), jnp.int32),
            pltpu.VMEM((SCH, CH), jnp.int32),
            pltpu.VMEM((SCH, CH), jnp.float32),
            pltpu.VMEM((CH, D), jnp.float32),
            pltpu.VMEM((CH, D), jnp.float32),
            pltpu.VMEM((HS,), jnp.float32),
            pltpu.VMEM_SHARED((N, D), jnp.float32),
            pltpu.VMEM_SHARED((NP,), jnp.float32),
            pltpu.SemaphoreType.DMA,
            pltpu.SemaphoreType.DMA,
            pltpu.SemaphoreType.DMA,
            pltpu.SemaphoreType.DMA,
        ],
        name="sc_edge_aggr",
    )


_BN = 1000


def _bf(x):
    return x.astype(jnp.bfloat16)


def _bdot(a, b):
    return jnp.dot(_bf(a), _bf(b), preferred_element_type=jnp.float32)


def _layer1_body(nf, hn0, hn1, he0, he1, wn, wh, we, b, o):
    hn = hn0[...] + hn1[...]
    he = he0[...] + he1[...]
    acc = _bdot(nf[...], wn[...]) + _bdot(hn, wh[...])
    acc = acc + _bf(he).astype(jnp.float32) * _bf(we[...]).astype(jnp.float32) + b[...]
    o[...] = jnp.maximum(acc, 0.0)


def _head_body(h, hn0, hn1, he0, he1, wh, wn, we, b1, w2, b2, w3, b3, o):
    hn = hn0[...] + hn1[...]
    he = he0[...] + he1[...]
    t = _bdot(h[...], wh[...]) + _bdot(hn, wn[...])
    t = t + _bf(he).astype(jnp.float32) * _bf(we[...]).astype(jnp.float32) + b1[...]
    t = jnp.maximum(t, 0.0)
    t = jnp.maximum(_bdot(t, w2[...]) + b2[...], 0.0)
    tb = _bf(t).astype(jnp.float32) * _bf(w3[...]).astype(jnp.float32)
    o[...] = jnp.sum(tb, axis=1, keepdims=True) + b3[...]


def _row_spec(width):
    return pl.BlockSpec((_BN, width), lambda i: (i, 0))


def _full_spec(shape):
    return pl.BlockSpec(shape, lambda i: tuple(0 for _ in shape))


def kernel(node_feat, edge_feat, edge_index, W_c1, b_c1, W_r1, b_r1, W_r2, b_r2, W_r3, b_r3):
    src3 = edge_index[0].reshape(NW * NSC, SCH, CH)
    dst3 = edge_index[1].reshape(NW * NSC, SCH, CH)
    ef3 = edge_feat.reshape(NW * NSC, SCH, CH)
    z2 = jnp.zeros((ZR, D), jnp.float32)
    z1 = jnp.zeros((NP,), jnp.float32)

    hn_p, he_a, he_b = _make_sc_aggr(True)(node_feat, src3, dst3, ef3, z2, z1)
    he0 = he_a[:N].reshape(N, 1)
    he1 = he_b[:N].reshape(N, 1)

    h = pl.pallas_call(
        _layer1_body,
        grid=(N // _BN,),
        in_specs=[
            _row_spec(D), _row_spec(D), _row_spec(D), _row_spec(1), _row_spec(1),
            _full_spec((D, H)), _full_spec((D, H)), _full_spec((1, H)), _full_spec((1, H)),
        ],
        out_specs=_row_spec(H),
        out_shape=jax.ShapeDtypeStruct((N, H), jnp.float32),
    )(
        node_feat, hn_p[0], hn_p[1], he0, he1,
        W_c1[:, :D].T, W_c1[:, D:2 * D].T, W_c1[:, 2 * D:].T, b_c1.reshape(1, H),
    )

    hn2_p, _, _ = _make_sc_aggr(False)(h, src3, dst3, ef3, z2, z1)

    out = pl.pallas_call(
        _head_body,
        grid=(N // _BN,),
        in_specs=[
            _row_spec(H), _row_spec(H), _row_spec(H), _row_spec(1), _row_spec(1),
            _full_spec((H, H)), _full_spec((H, H)), _full_spec((1, H)), _full_spec((1, H)),
            _full_spec((H, H)), _full_spec((1, H)),
            _full_spec((1, H)), _full_spec((1, 1)),
        ],
        out_specs=_row_spec(1),
        out_shape=jax.ShapeDtypeStruct((N, 1), jnp.float32),
    )(
        h, hn2_p[0], hn2_p[1], he0, he1,
        W_r1[:, :H].T, W_r1[:, H:2 * H].T, W_r1[:, 2 * H:].T, b_r1.reshape(1, H),
        W_r2.T, b_r2.reshape(1, H),
        W_r3, b_r3.reshape(1, 1),
    )
    return out


# speedup vs baseline: 1.1720x; 1.1720x over previous
import functools

import jax
import jax.numpy as jnp
from jax import lax
from jax.experimental import pallas as pl
from jax.experimental.pallas import tpu as pltpu
from jax.experimental.pallas import tpu_sc as plsc

N = 10000
E = 320000
D = 128
H = 128

NC = 2
NS = 16
NW = NC * NS
EPW = E // NW
CH = 80
SCH = 25
NSC = EPW // (CH * SCH)
ZR = N // 10
NP = 10240
HS = NP // 10

_MESH = plsc.VectorSubcoreMesh(
    core_axis_name="c", subcore_axis_name="s", num_cores=NC, num_subcores=NS
)


def _sc_aggr_body(with_ef, feat_hbm, src_hbm, dst_hbm, ef_hbm, z2_hbm, z1_hbm,
                  hn_out, he0_out, he1_out, src_v, dst_v, ef_v, rows0, rows1,
                  he_v, acc_sh, he_sh, g0, g1, s0, s1):
    c = lax.axis_index("c")
    s = lax.axis_index("s")
    wid = s * NC + c

    @pl.when(s < 10)
    def _zero():
        pltpu.sync_copy(z2_hbm, acc_sh.at[pl.ds(s * ZR, ZR)])
        if with_ef:
            pltpu.sync_copy(z1_hbm.at[pl.ds(s * HS, HS)], he_v)
            pltpu.sync_copy(he_v, he_sh.at[pl.ds(s * HS, HS)])

    plsc.subcore_barrier()

    def gather_start(j, buf, sem):
        pltpu.make_async_copy(feat_hbm.at[src_v.at[j]], buf, sem).start()

    def gather_wait(j, buf, sem):
        pltpu.make_async_copy(feat_hbm.at[src_v.at[j]], buf, sem).wait()

    def scatter_start(j, buf, sem):
        pltpu.async_copy(buf, acc_sh.at[dst_v.at[j]], sem, add=True)

    def scatter_wait(j, buf, sem):
        pltpu.make_async_copy(buf, acc_sh.at[dst_v.at[j]], sem).wait()

    def he_scatter(j):
        if with_ef:
            pltpu.sync_copy(ef_v.at[j], he_sh.at[dst_v.at[j]], add=True)

    def super_step(k, carry):
        pltpu.sync_copy(src_hbm.at[wid * NSC + k], src_v)
        pltpu.sync_copy(dst_hbm.at[wid * NSC + k], dst_v)
        if with_ef:
            pltpu.sync_copy(ef_hbm.at[wid * NSC + k], ef_v)

        gather_start(0, rows0, g0)

        def pair(jj, inner):
            j0 = 2 * jj
            gather_start(j0 + 1, rows1, g1)
            gather_wait(j0, rows0, g0)
            pltpu.sync_copy(rows0, acc_sh.at[dst_v.at[j0]], add=True)
            he_scatter(j0)
            gather_start(j0 + 2, rows0, g0)
            gather_wait(j0 + 1, rows1, g1)
            pltpu.sync_copy(rows1, acc_sh.at[dst_v.at[j0 + 1]], add=True)
            he_scatter(j0 + 1)
            return inner

        lax.fori_loop(0, SCH // 2, pair, 0)
        gather_wait(SCH - 1, rows0, g0)
        pltpu.sync_copy(rows0, acc_sh.at[dst_v.at[SCH - 1]], add=True)
        he_scatter(SCH - 1)
        return carry

    lax.fori_loop(0, NSC, super_step, 0)
    plsc.subcore_barrier()

    @pl.when(s < 10)
    def _writeback():
        pltpu.sync_copy(acc_sh.at[pl.ds(s * ZR, ZR)], hn_out.at[c, pl.ds(s * ZR, ZR)])
        if with_ef:
            pltpu.sync_copy(he_sh.at[pl.ds(s * HS, HS)], he_v)

            @pl.when(c == 0)
            def _wb_he0():
                pltpu.sync_copy(he_v, he0_out.at[pl.ds(s * HS, HS)])

            @pl.when(c == 1)
            def _wb_he1():
                pltpu.sync_copy(he_v, he1_out.at[pl.ds(s * HS, HS)])


def _make_sc_aggr(with_ef):
    return pl.kernel(
        functools.partial(_sc_aggr_body, with_ef),
        out_type=[
            jax.ShapeDtypeStruct((NC, N, D), jnp.float32),
            jax.ShapeDtypeStruct((NP,), jnp.float32),
            jax.ShapeDtypeStruct((NP,), jnp.float32),
        ],
        mesh=_MESH,
        scratch_types=[
            pltpu.VMEM((SCH, CH), jnp.int32),
            pltpu.VMEM((SCH, CH), jnp.int32),
            pltpu.VMEM((SCH, CH), jnp.float32),
            pltpu.VMEM((CH, D), jnp.float32),
            pltpu.VMEM((CH, D), jnp.float32),
            pltpu.VMEM((HS,), jnp.float32),
            pltpu.VMEM_SHARED((N, D), jnp.float32),
            pltpu.VMEM_SHARED((NP,), jnp.float32),
            pltpu.SemaphoreType.DMA,
            pltpu.SemaphoreType.DMA,
            pltpu.SemaphoreType.DMA,
            pltpu.SemaphoreType.DMA,
        ],
        name="sc_edge_aggr",
    )


_BN = 1000


def _bf(x):
    return x.astype(jnp.bfloat16)


def _bdot(a, b):
    return jnp.dot(_bf(a), _bf(b), preferred_element_type=jnp.float32)


def _layer1_body(nf, hn0, hn1, he0, he1, wn, wh, we, b, o):
    hn = hn0[...] + hn1[...]
    he = he0[...] + he1[...]
    acc = _bdot(nf[...], wn[...]) + _bdot(hn, wh[...])
    acc = acc + _bf(he).astype(jnp.float32) * _bf(we[...]).astype(jnp.float32) + b[...]
    o[...] = jnp.maximum(acc, 0.0)


def _head_body(h, hn0, hn1, he0, he1, wh, wn, we, b1, w2, b2, w3, b3, o):
    hn = hn0[...] + hn1[...]
    he = he0[...] + he1[...]
    t = _bdot(h[...], wh[...]) + _bdot(hn, wn[...])
    t = t + _bf(he).astype(jnp.float32) * _bf(we[...]).astype(jnp.float32) + b1[...]
    t = jnp.maximum(t, 0.0)
    t = jnp.maximum(_bdot(t, w2[...]) + b2[...], 0.0)
    tb = _bf(t).astype(jnp.float32) * _bf(w3[...]).astype(jnp.float32)
    o[...] = jnp.sum(tb, axis=1, keepdims=True) + b3[...]


def _row_spec(width):
    return pl.BlockSpec((_BN, width), lambda i: (i, 0))


def _full_spec(shape):
    return pl.BlockSpec(shape, lambda i: tuple(0 for _ in shape))


def kernel(node_feat, edge_feat, edge_index, W_c1, b_c1, W_r1, b_r1, W_r2, b_r2, W_r3, b_r3):
    src3 = edge_index[0].reshape(NW * NSC, SCH, CH)
    dst3 = edge_index[1].reshape(NW * NSC, SCH, CH)
    ef3 = edge_feat.reshape(NW * NSC, SCH, CH)
    z2 = jnp.zeros((ZR, D), jnp.float32)
    z1 = jnp.zeros((NP,), jnp.float32)

    hn_p, he_a, he_b = _make_sc_aggr(True)(node_feat, src3, dst3, ef3, z2, z1)
    he0 = he_a[:N].reshape(N, 1)
    he1 = he_b[:N].reshape(N, 1)

    h = pl.pallas_call(
        _layer1_body,
        grid=(N // _BN,),
        in_specs=[
            _row_spec(D), _row_spec(D), _row_spec(D), _row_spec(1), _row_spec(1),
            _full_spec((D, H)), _full_spec((D, H)), _full_spec((1, H)), _full_spec((1, H)),
        ],
        out_specs=_row_spec(H),
        out_shape=jax.ShapeDtypeStruct((N, H), jnp.float32),
    )(
        node_feat, hn_p[0], hn_p[1], he0, he1,
        W_c1[:, :D].T, W_c1[:, D:2 * D].T, W_c1[:, 2 * D:].T, b_c1.reshape(1, H),
    )

    hn2_p, _, _ = _make_sc_aggr(False)(h, src3, dst3, ef3, z2, z1)

    out = pl.pallas_call(
        _head_body,
        grid=(N // _BN,),
        in_specs=[
            _row_spec(H), _row_spec(H), _row_spec(H), _row_spec(1), _row_spec(1),
            _full_spec((H, H)), _full_spec((H, H)), _full_spec((1, H)), _full_spec((1, H)),
            _full_spec((H, H)), _full_spec((1, H)),
            _full_spec((1, H)), _full_spec((1, 1)),
        ],
        out_specs=_row_spec(1),
        out_shape=jax.ShapeDtypeStruct((N, 1), jnp.float32),
    )(
        h, hn2_p[0], hn2_p[1], he0, he1,
        W_r1[:, :H].T, W_r1[:, H:2 * H].T, W_r1[:, 2 * H:].T, b_r1.reshape(1, H),
        W_r2.T, b_r2.reshape(1, H),
        W_r3, b_r3.reshape(1, 1),
    )
    return out


# speedup vs baseline: 1.3177x; 1.1243x over previous
import functools

import jax
import jax.numpy as jnp
from jax import lax
from jax.experimental import pallas as pl
from jax.experimental.pallas import tpu as pltpu
from jax.experimental.pallas import tpu_sc as plsc

N = 10000
E = 320000
D = 128
H = 128

NC = 2
NS = 16
NW = NC * NS
EPW = E // NW
CH = 80
SCH = 25
NSC = EPW // (CH * SCH)
ZR = N // 10
NP = 10240
HS = NP // 10

_MESH = plsc.VectorSubcoreMesh(
    core_axis_name="c", subcore_axis_name="s", num_cores=NC, num_subcores=NS
)


def _sc_aggr_body(with_ef, feat_hbm, src_hbm, dst_hbm, ef_hbm, z2_hbm, z1_hbm,
                  hn_out, he0_out, he1_out, src_v, dst_v, ef_v, rows0, rows1,
                  rows2, he_v, acc_sh, he_sh, g0, g1, g2):
    c = lax.axis_index("c")
    s = lax.axis_index("s")
    wid = s * NC + c

    @pl.when(s < 10)
    def _zero():
        pltpu.sync_copy(z2_hbm, acc_sh.at[pl.ds(s * ZR, ZR)])
        if with_ef:
            pltpu.sync_copy(z1_hbm.at[pl.ds(s * HS, HS)], he_v)
            pltpu.sync_copy(he_v, he_sh.at[pl.ds(s * HS, HS)])

    plsc.subcore_barrier()

    def gather_start(j, buf, sem):
        pltpu.make_async_copy(feat_hbm.at[src_v.at[j]], buf, sem).start()

    def gather_wait(j, buf, sem):
        pltpu.make_async_copy(feat_hbm.at[src_v.at[j]], buf, sem).wait()

    def he_scatter(j):
        if with_ef:
            pltpu.sync_copy(ef_v.at[j], he_sh.at[dst_v.at[j]], add=True)

    def super_step(k, carry):
        pltpu.sync_copy(src_hbm.at[wid * NSC + k], src_v)
        pltpu.sync_copy(dst_hbm.at[wid * NSC + k], dst_v)
        if with_ef:
            pltpu.sync_copy(ef_hbm.at[wid * NSC + k], ef_v)

        bufs = (rows0, rows1, rows2)
        sems = (g0, g1, g2)

        def do_chunk(j, u):
            gather_wait(j, bufs[u], sems[u])
            pltpu.sync_copy(bufs[u], acc_sh.at[dst_v.at[j]], add=True)
            he_scatter(j)

        gather_start(0, rows0, g0)
        gather_start(1, rows1, g1)

        def tri(t, inner):
            j0 = 3 * t
            for u in range(3):
                gather_start(j0 + u + 2, bufs[(u + 2) % 3], sems[(u + 2) % 3])
                do_chunk(j0 + u, u)
            return inner

        lax.fori_loop(0, SCH // 3 - 1, tri, 0)
        gather_start(SCH - 2, rows2, g2)
        do_chunk(SCH - 4, 0)
        gather_start(SCH - 1, rows0, g0)
        do_chunk(SCH - 3, 1)
        do_chunk(SCH - 2, 2)
        gather_wait(SCH - 1, rows0, g0)
        pltpu.sync_copy(rows0, acc_sh.at[dst_v.at[SCH - 1]], add=True)
        he_scatter(SCH - 1)
        return carry

    lax.fori_loop(0, NSC, super_step, 0)
    plsc.subcore_barrier()

    @pl.when(s < 10)
    def _writeback():
        pltpu.sync_copy(acc_sh.at[pl.ds(s * ZR, ZR)], hn_out.at[c, pl.ds(s * ZR, ZR)])
        if with_ef:
            pltpu.sync_copy(he_sh.at[pl.ds(s * HS, HS)], he_v)

            @pl.when(c == 0)
            def _wb_he0():
                pltpu.sync_copy(he_v, he0_out.at[pl.ds(s * HS, HS)])

            @pl.when(c == 1)
            def _wb_he1():
                pltpu.sync_copy(he_v, he1_out.at[pl.ds(s * HS, HS)])


def _make_sc_aggr(with_ef):
    return pl.kernel(
        functools.partial(_sc_aggr_body, with_ef),
        out_type=[
            jax.ShapeDtypeStruct((NC, N, D), jnp.float32),
            jax.ShapeDtypeStruct((NP,), jnp.float32),
            jax.ShapeDtypeStruct((NP,), jnp.float32),
        ],
        mesh=_MESH,
        scratch_types=[
            pltpu.VMEM((SCH, CH), jnp.int32),
            pltpu.VMEM((SCH, CH), jnp.int32),
            pltpu.VMEM((SCH, CH), jnp.float32),
            pltpu.VMEM((CH, D), jnp.float32),
            pltpu.VMEM((CH, D), jnp.float32),
            pltpu.VMEM((CH, D), jnp.float32),
            pltpu.VMEM((HS,), jnp.float32),
            pltpu.VMEM_SHARED((N, D), jnp.float32),
            pltpu.VMEM_SHARED((NP,), jnp.float32),
            pltpu.SemaphoreType.DMA,
            pltpu.SemaphoreType.DMA,
            pltpu.SemaphoreType.DMA,
        ],
        name="sc_edge_aggr",
    )


_BN = 1000


def _bf(x):
    return x.astype(jnp.bfloat16)


def _bdot(a, b):
    return jnp.dot(_bf(a), _bf(b), preferred_element_type=jnp.float32)


def _layer1_body(nf, hn0, hn1, he0, he1, wn, wh, we, b, o):
    hn = hn0[...] + hn1[...]
    he = he0[...] + he1[...]
    acc = _bdot(nf[...], wn[...]) + _bdot(hn, wh[...])
    acc = acc + _bf(he).astype(jnp.float32) * _bf(we[...]).astype(jnp.float32) + b[...]
    o[...] = jnp.maximum(acc, 0.0)


def _head_body(h, hn0, hn1, he0, he1, wh, wn, we, b1, w2, b2, w3, b3, o):
    hn = hn0[...] + hn1[...]
    he = he0[...] + he1[...]
    t = _bdot(h[...], wh[...]) + _bdot(hn, wn[...])
    t = t + _bf(he).astype(jnp.float32) * _bf(we[...]).astype(jnp.float32) + b1[...]
    t = jnp.maximum(t, 0.0)
    t = jnp.maximum(_bdot(t, w2[...]) + b2[...], 0.0)
    tb = _bf(t).astype(jnp.float32) * _bf(w3[...]).astype(jnp.float32)
    o[...] = jnp.sum(tb, axis=1, keepdims=True) + b3[...]


def _row_spec(width):
    return pl.BlockSpec((_BN, width), lambda i: (i, 0))


def _full_spec(shape):
    return pl.BlockSpec(shape, lambda i: tuple(0 for _ in shape))


def kernel(node_feat, edge_feat, edge_index, W_c1, b_c1, W_r1, b_r1, W_r2, b_r2, W_r3, b_r3):
    src3 = edge_index[0].reshape(NW * NSC, SCH, CH)
    dst3 = edge_index[1].reshape(NW * NSC, SCH, CH)
    ef3 = edge_feat.reshape(NW * NSC, SCH, CH)
    z2 = jnp.zeros((ZR, D), jnp.float32)
    z1 = jnp.zeros((NP,), jnp.float32)

    hn_p, he_a, he_b = _make_sc_aggr(True)(node_feat, src3, dst3, ef3, z2, z1)
    he0 = he_a[:N].reshape(N, 1)
    he1 = he_b[:N].reshape(N, 1)

    h = pl.pallas_call(
        _layer1_body,
        grid=(N // _BN,),
        in_specs=[
            _row_spec(D), _row_spec(D), _row_spec(D), _row_spec(1), _row_spec(1),
            _full_spec((D, H)), _full_spec((D, H)), _full_spec((1, H)), _full_spec((1, H)),
        ],
        out_specs=_row_spec(H),
        out_shape=jax.ShapeDtypeStruct((N, H), jnp.float32),
    )(
        node_feat, hn_p[0], hn_p[1], he0, he1,
        W_c1[:, :D].T, W_c1[:, D:2 * D].T, W_c1[:, 2 * D:].T, b_c1.reshape(1, H),
    )

    hn2_p, _, _ = _make_sc_aggr(False)(h, src3, dst3, ef3, z2, z1)

    out = pl.pallas_call(
        _head_body,
        grid=(N // _BN,),
        in_specs=[
            _row_spec(H), _row_spec(H), _row_spec(H), _row_spec(1), _row_spec(1),
            _full_spec((H, H)), _full_spec((H, H)), _full_spec((1, H)), _full_spec((1, H)),
            _full_spec((H, H)), _full_spec((1, H)),
            _full_spec((1, H)), _full_spec((1, 1)),
        ],
        out_specs=_row_spec(1),
        out_shape=jax.ShapeDtypeStruct((N, 1), jnp.float32),
    )(
        h, hn2_p[0], hn2_p[1], he0, he1,
        W_r1[:, :H].T, W_r1[:, H:2 * H].T, W_r1[:, 2 * H:].T, b_r1.reshape(1, H),
        W_r2.T, b_r2.reshape(1, H),
        W_r3, b_r3.reshape(1, 1),
    )
    return out


# speedup vs baseline: 1.3283x; 1.0080x over previous
import functools

import jax
import jax.numpy as jnp
from jax import lax
from jax.experimental import pallas as pl
from jax.experimental.pallas import tpu as pltpu
from jax.experimental.pallas import tpu_sc as plsc

N = 10000
E = 320000
D = 128
H = 128

NC = 2
NS = 16
NW = NC * NS
EPW = E // NW
CH = 80
SCH = 25
NSC = EPW // (CH * SCH)
ZR = N // 10
NP = 10240
HS = NP // 10

_MESH = plsc.VectorSubcoreMesh(
    core_axis_name="c", subcore_axis_name="s", num_cores=NC, num_subcores=NS
)


def _sc_aggr_body(with_ef, feat_hbm, src_hbm, dst_hbm, ef_hbm, z2_hbm, z1_hbm,
                  hn_out, he0_out, he1_out, src_v, dst_v, ef_v, rows0, rows1,
                  rows2, he_v, acc_sh, he_sh, g0, g1, g2):
    c = lax.axis_index("c")
    s = lax.axis_index("s")
    wid = s * NC + c

    @pl.when(s < 10)
    def _zero():
        pltpu.sync_copy(z2_hbm, acc_sh.at[pl.ds(s * ZR, ZR)])
        if with_ef:
            pltpu.sync_copy(z1_hbm.at[pl.ds(s * HS, HS)], he_v)
            pltpu.sync_copy(he_v, he_sh.at[pl.ds(s * HS, HS)])

    plsc.subcore_barrier()

    def gather_start(j, buf, sem):
        pltpu.make_async_copy(feat_hbm.at[src_v.at[j]], buf, sem).start()

    def gather_wait(j, buf, sem):
        pltpu.make_async_copy(feat_hbm.at[src_v.at[j]], buf, sem).wait()

    def he_scatter(j):
        if with_ef:
            pltpu.sync_copy(ef_v.at[j], he_sh.at[dst_v.at[j]], add=True)

    def super_step(k, carry):
        pltpu.sync_copy(src_hbm.at[wid * NSC + k], src_v)
        pltpu.sync_copy(dst_hbm.at[wid * NSC + k], dst_v)
        if with_ef:
            pltpu.sync_copy(ef_hbm.at[wid * NSC + k], ef_v)

        bufs = (rows0, rows1, rows2)
        sems = (g0, g1, g2)

        def do_chunk(j, u):
            gather_wait(j, bufs[u], sems[u])
            pltpu.sync_copy(bufs[u], acc_sh.at[dst_v.at[j]], add=True)
            he_scatter(j)

        gather_start(0, rows0, g0)
        gather_start(1, rows1, g1)

        def tri(t, inner):
            j0 = 3 * t
            for u in range(3):
                gather_start(j0 + u + 2, bufs[(u + 2) % 3], sems[(u + 2) % 3])
                do_chunk(j0 + u, u)
            return inner

        lax.fori_loop(0, SCH // 3 - 1, tri, 0)
        gather_start(SCH - 2, rows2, g2)
        do_chunk(SCH - 4, 0)
        gather_start(SCH - 1, rows0, g0)
        do_chunk(SCH - 3, 1)
        do_chunk(SCH - 2, 2)
        gather_wait(SCH - 1, rows0, g0)
        pltpu.sync_copy(rows0, acc_sh.at[dst_v.at[SCH - 1]], add=True)
        he_scatter(SCH - 1)
        return carry

    lax.fori_loop(0, NSC, super_step, 0)
    plsc.subcore_barrier()

    @pl.when(s < 10)
    def _writeback():
        pltpu.sync_copy(acc_sh.at[pl.ds(s * ZR, ZR)], hn_out.at[c, pl.ds(s * ZR, ZR)])
        if with_ef:
            pltpu.sync_copy(he_sh.at[pl.ds(s * HS, HS)], he_v)

            @pl.when(c == 0)
            def _wb_he0():
                pltpu.sync_copy(he_v, he0_out.at[pl.ds(s * HS, HS)])

            @pl.when(c == 1)
            def _wb_he1():
                pltpu.sync_copy(he_v, he1_out.at[pl.ds(s * HS, HS)])


def _make_sc_aggr(with_ef):
    return pl.kernel(
        functools.partial(_sc_aggr_body, with_ef),
        out_type=[
            jax.ShapeDtypeStruct((NC, N, D), jnp.float32),
            jax.ShapeDtypeStruct((NP,), jnp.float32),
            jax.ShapeDtypeStruct((NP,), jnp.float32),
        ],
        mesh=_MESH,
        scratch_types=[
            pltpu.VMEM((SCH, CH), jnp.int32),
            pltpu.VMEM((SCH, CH), jnp.int32),
            pltpu.VMEM((SCH, CH), jnp.float32),
            pltpu.VMEM((CH, D), jnp.float32),
            pltpu.VMEM((CH, D), jnp.float32),
            pltpu.VMEM((CH, D), jnp.float32),
            pltpu.VMEM((HS,), jnp.float32),
            pltpu.VMEM_SHARED((N, D), jnp.float32),
            pltpu.VMEM_SHARED((NP,), jnp.float32),
            pltpu.SemaphoreType.DMA,
            pltpu.SemaphoreType.DMA,
            pltpu.SemaphoreType.DMA,
        ],
        name="sc_edge_aggr",
    )


_BN = 2000


def _bf(x):
    return x.astype(jnp.bfloat16)


def _bdot(a, w):
    return jnp.dot(_bf(a), w, preferred_element_type=jnp.float32)


def _layer1_body(nf, hn0, hn1, he0, he1, wn, wh, we, b, o):
    hn = hn0[...] + hn1[...]
    he = he0[...] + he1[...]
    acc = _bdot(nf[...], wn[...]) + _bdot(hn, wh[...])
    acc = acc + _bf(he).astype(jnp.float32) * we[...].astype(jnp.float32) + b[...]
    o[...] = jnp.maximum(acc, 0.0)


def _head_body(h, hn0, hn1, he0, he1, wh, wn, we, b1, w2, b2, w3, b3, o):
    hn = hn0[...] + hn1[...]
    he = he0[...] + he1[...]
    t = _bdot(h[...], wh[...]) + _bdot(hn, wn[...])
    t = t + _bf(he).astype(jnp.float32) * we[...].astype(jnp.float32) + b1[...]
    t = jnp.maximum(t, 0.0)
    t = jnp.maximum(_bdot(t, w2[...]) + b2[...], 0.0)
    tb = _bf(t).astype(jnp.float32) * w3[...].astype(jnp.float32)
    o[...] = jnp.sum(tb, axis=1, keepdims=True) + b3[...]


def _row_spec(width):
    return pl.BlockSpec((_BN, width), lambda i: (i, 0))


def _full_spec(shape):
    return pl.BlockSpec(shape, lambda i: tuple(0 for _ in shape))


def kernel(node_feat, edge_feat, edge_index, W_c1, b_c1, W_r1, b_r1, W_r2, b_r2, W_r3, b_r3):
    src3 = edge_index[0].reshape(NW * NSC, SCH, CH)
    dst3 = edge_index[1].reshape(NW * NSC, SCH, CH)
    ef3 = edge_feat.reshape(NW * NSC, SCH, CH)
    z2 = jnp.zeros((ZR, D), jnp.float32)
    z1 = jnp.zeros((NP,), jnp.float32)

    hn_p, he_a, he_b = _make_sc_aggr(True)(node_feat, src3, dst3, ef3, z2, z1)
    he0 = he_a[:N].reshape(N, 1)
    he1 = he_b[:N].reshape(N, 1)

    h = pl.pallas_call(
        _layer1_body,
        grid=(N // _BN,),
        in_specs=[
            _row_spec(D), _row_spec(D), _row_spec(D), _row_spec(1), _row_spec(1),
            _full_spec((D, H)), _full_spec((D, H)), _full_spec((1, H)), _full_spec((1, H)),
        ],
        out_specs=_row_spec(H),
        out_shape=jax.ShapeDtypeStruct((N, H), jnp.float32),
    )(
        node_feat, hn_p[0], hn_p[1], he0, he1,
        _bf(W_c1[:, :D].T), _bf(W_c1[:, D:2 * D].T),
        _bf(W_c1[:, 2 * D:].T).astype(jnp.float32),
        b_c1.reshape(1, H),
    )

    hn2_p, _, _ = _make_sc_aggr(False)(h, src3, dst3, ef3, z2, z1)

    out = pl.pallas_call(
        _head_body,
        grid=(N // _BN,),
        in_specs=[
            _row_spec(H), _row_spec(H), _row_spec(H), _row_spec(1), _row_spec(1),
            _full_spec((H, H)), _full_spec((H, H)), _full_spec((1, H)), _full_spec((1, H)),
            _full_spec((H, H)), _full_spec((1, H)),
            _full_spec((1, H)), _full_spec((1, 1)),
        ],
        out_specs=_row_spec(1),
        out_shape=jax.ShapeDtypeStruct((N, 1), jnp.float32),
    )(
        h, hn2_p[0], hn2_p[1], he0, he1,
        _bf(W_r1[:, :H].T), _bf(W_r1[:, H:2 * H].T),
        _bf(W_r1[:, 2 * H:].T).astype(jnp.float32),
        b_r1.reshape(1, H),
        _bf(W_r2.T), b_r2.reshape(1, H),
        _bf(W_r3).astype(jnp.float32), b_r3.reshape(1, 1),
    )
    return out


# speedup vs baseline: 1.3283x; 1.0000x over previous
import functools

import jax
import jax.numpy as jnp
from jax import lax
from jax.experimental import pallas as pl
from jax.experimental.pallas import tpu as pltpu
from jax.experimental.pallas import tpu_sc as plsc

N = 10000
E = 320000
D = 128
H = 128

NC = 2
NS = 16
NW = NC * NS
EPW = E // NW
CH = 80
SCH = 25
NSC = EPW // (CH * SCH)
ZR = N // 10
NP = 10240
HS = NP // 10

_MESH = plsc.VectorSubcoreMesh(
    core_axis_name="c", subcore_axis_name="s", num_cores=NC, num_subcores=NS
)


def _sc_aggr_body(with_ef, feat_hbm, src_hbm, dst_hbm, ef_hbm, z2_hbm, z1_hbm,
                  hn_out, he0_out, he1_out, src_v, dst_v, ef_v, rows0, rows1,
                  rows2, he_v, acc_sh, he_sh, g0, g1, g2):
    c = lax.axis_index("c")
    s = lax.axis_index("s")
    wid = s * NC + c

    @pl.when(s < 10)
    def _zero():
        pltpu.sync_copy(z2_hbm, acc_sh.at[pl.ds(s * ZR, ZR)])
        if with_ef:
            pltpu.sync_copy(z1_hbm.at[pl.ds(s * HS, HS)], he_v)
            pltpu.sync_copy(he_v, he_sh.at[pl.ds(s * HS, HS)])

    plsc.subcore_barrier()

    def gather_start(j, buf, sem):
        pltpu.make_async_copy(feat_hbm.at[src_v.at[j]], buf, sem).start()

    def gather_wait(j, buf, sem):
        pltpu.make_async_copy(feat_hbm.at[src_v.at[j]], buf, sem).wait()

    def he_scatter(j):
        if with_ef:
            pltpu.sync_copy(ef_v.at[j], he_sh.at[dst_v.at[j]], add=True)

    def super_step(k, carry):
        pltpu.sync_copy(src_hbm.at[wid * NSC + k], src_v)
        pltpu.sync_copy(dst_hbm.at[wid * NSC + k], dst_v)
        if with_ef:
            pltpu.sync_copy(ef_hbm.at[wid * NSC + k], ef_v)

        bufs = (rows0, rows1, rows2)
        sems = (g0, g1, g2)

        def do_chunk(j, u):
            gather_wait(j, bufs[u], sems[u])
            pltpu.sync_copy(bufs[u], acc_sh.at[dst_v.at[j]], add=True)
            he_scatter(j)

        gather_start(0, rows0, g0)
        gather_start(1, rows1, g1)

        def tri(t, inner):
            j0 = 3 * t
            for u in range(3):
                gather_start(j0 + u + 2, bufs[(u + 2) % 3], sems[(u + 2) % 3])
                do_chunk(j0 + u, u)
            return inner

        lax.fori_loop(0, SCH // 3 - 1, tri, 0)
        gather_start(SCH - 2, rows2, g2)
        do_chunk(SCH - 4, 0)
        gather_start(SCH - 1, rows0, g0)
        do_chunk(SCH - 3, 1)
        do_chunk(SCH - 2, 2)
        do_chunk(SCH - 1, 0)
        return carry

    lax.fori_loop(0, NSC, super_step, 0)
    plsc.subcore_barrier()

    @pl.when(s < 10)
    def _writeback():
        pltpu.sync_copy(acc_sh.at[pl.ds(s * ZR, ZR)], hn_out.at[c, pl.ds(s * ZR, ZR)])
        if with_ef:
            pltpu.sync_copy(he_sh.at[pl.ds(s * HS, HS)], he_v)

            @pl.when(c == 0)
            def _wb_he0():
                pltpu.sync_copy(he_v, he0_out.at[pl.ds(s * HS, HS)])

            @pl.when(c == 1)
            def _wb_he1():
                pltpu.sync_copy(he_v, he1_out.at[pl.ds(s * HS, HS)])


def _make_sc_aggr(with_ef):
    return pl.kernel(
        functools.partial(_sc_aggr_body, with_ef),
        out_type=[
            jax.ShapeDtypeStruct((NC, N, D), jnp.float32),
            jax.ShapeDtypeStruct((NP,), jnp.float32),
            jax.ShapeDtypeStruct((NP,), jnp.float32),
        ],
        mesh=_MESH,
        scratch_types=[
            pltpu.VMEM((SCH, CH), jnp.int32),
            pltpu.VMEM((SCH, CH), jnp.int32),
            pltpu.VMEM((SCH, CH), jnp.float32),
            pltpu.VMEM((CH, D), jnp.float32),
            pltpu.VMEM((CH, D), jnp.float32),
            pltpu.VMEM((CH, D), jnp.float32),
            pltpu.VMEM((HS,), jnp.float32),
            pltpu.VMEM_SHARED((N, D), jnp.float32),
            pltpu.VMEM_SHARED((NP,), jnp.float32),
            pltpu.SemaphoreType.DMA,
            pltpu.SemaphoreType.DMA,
            pltpu.SemaphoreType.DMA,
        ],
        name="sc_edge_aggr",
    )


_BN = 2000


def _bf(x):
    return x.astype(jnp.bfloat16)


def _bdot(a, w):
    return jnp.dot(_bf(a), w, preferred_element_type=jnp.float32)


def _layer1_body(nf, hn0, hn1, he0, he1, wn, wh, we, b, o):
    hn = hn0[...] + hn1[...]
    he = he0[...] + he1[...]
    acc = _bdot(nf[...], wn[...]) + _bdot(hn, wh[...])
    acc = acc + _bf(he).astype(jnp.float32) * we[...].astype(jnp.float32) + b[...]
    o[...] = jnp.maximum(acc, 0.0)


def _head_body(h, hn0, hn1, he0, he1, wh, wn, we, b1, w2, b2, w3, b3, o):
    hn = hn0[...] + hn1[...]
    he = he0[...] + he1[...]
    t = _bdot(h[...], wh[...]) + _bdot(hn, wn[...])
    t = t + _bf(he).astype(jnp.float32) * we[...].astype(jnp.float32) + b1[...]
    t = jnp.maximum(t, 0.0)
    t = jnp.maximum(_bdot(t, w2[...]) + b2[...], 0.0)
    tb = _bf(t).astype(jnp.float32) * w3[...].astype(jnp.float32)
    o[...] = jnp.sum(tb, axis=1, keepdims=True) + b3[...]


def _row_spec(width):
    return pl.BlockSpec((_BN, width), lambda i: (i, 0))


def _full_spec(shape):
    return pl.BlockSpec(shape, lambda i: tuple(0 for _ in shape))


def kernel(node_feat, edge_feat, edge_index, W_c1, b_c1, W_r1, b_r1, W_r2, b_r2, W_r3, b_r3):
    src3 = edge_index[0].reshape(NW * NSC, SCH, CH)
    dst3 = edge_index[1].reshape(NW * NSC, SCH, CH)
    ef3 = edge_feat.reshape(NW * NSC, SCH, CH)
    z2 = jnp.zeros((ZR, D), jnp.float32)
    z1 = jnp.zeros((NP,), jnp.float32)

    hn_p, he_a, he_b = _make_sc_aggr(True)(node_feat, src3, dst3, ef3, z2, z1)
    he0 = he_a[:N].reshape(N, 1)
    he1 = he_b[:N].reshape(N, 1)

    h = pl.pallas_call(
        _layer1_body,
        grid=(N // _BN,),
        in_specs=[
            _row_spec(D), _row_spec(D), _row_spec(D), _row_spec(1), _row_spec(1),
            _full_spec((D, H)), _full_spec((D, H)), _full_spec((1, H)), _full_spec((1, H)),
        ],
        out_specs=_row_spec(H),
        out_shape=jax.ShapeDtypeStruct((N, H), jnp.float32),
    )(
        node_feat, hn_p[0], hn_p[1], he0, he1,
        _bf(W_c1[:, :D].T), _bf(W_c1[:, D:2 * D].T),
        _bf(W_c1[:, 2 * D:].T).astype(jnp.float32),
        b_c1.reshape(1, H),
    )

    hn2_p, _, _ = _make_sc_aggr(False)(h, src3, dst3, ef3, z2, z1)

    out = pl.pallas_call(
        _head_body,
        grid=(N // _BN,),
        in_specs=[
            _row_spec(H), _row_spec(H), _row_spec(H), _row_spec(1), _row_spec(1),
            _full_spec((H, H)), _full_spec((H, H)), _full_spec((1, H)), _full_spec((1, H)),
            _full_spec((H, H)), _full_spec((1, H)),
            _full_spec((1, H)), _full_spec((1, 1)),
        ],
        out_specs=_row_spec(1),
        out_shape=jax.ShapeDtypeStruct((N, 1), jnp.float32),
    )(
        h, hn2_p[0], hn2_p[1], he0, he1,
        _bf(W_r1[:, :H].T), _bf(W_r1[:, H:2 * H].T),
        _bf(W_r1[:, 2 * H:].T).astype(jnp.float32),
        b_r1.reshape(1, H),
        _bf(W_r2.T), b_r2.reshape(1, H),
        _bf(W_r3).astype(jnp.float32), b_r3.reshape(1, 1),
    )
    return out
